# baseline (speedup 1.0000x reference)
import jax
import jax.numpy as jnp
from jax.experimental import pallas as pl


def _make_tc(B, V, D, BM=512, BK=256):
    def body(idx_ref, table_ref, out_ref):
        idx = idx_ref[...]
        acc = jnp.zeros((BM, D), jnp.float32)
        for k0 in range(0, V, BK):
            kk = BK if k0 + BK <= V else V - k0
            col = jax.lax.broadcasted_iota(jnp.int32, (BM, kk), 1) + k0
            oh = (col == idx).astype(jnp.bfloat16)
            acc += jnp.dot(
                oh,
                table_ref[pl.ds(k0, kk), :].astype(jnp.bfloat16),
                preferred_element_type=jnp.float32,
            )
        out_ref[...] = acc

    def f(t, table):
        t2 = t.reshape(B, 1).astype(jnp.int32)
        return pl.pallas_call(
            body,
            grid=(B // BM,),
            in_specs=[
                pl.BlockSpec((BM, 1), lambda i: (i, 0)),
                pl.BlockSpec((V, D), lambda i: (0, 0)),
            ],
            out_specs=pl.BlockSpec((BM, D), lambda i: (i, 0)),
            out_shape=jax.ShapeDtypeStruct((B, D), jnp.float32),
        )(t2, table)

    return f


def kernel(t, table):
    (B,) = t.shape
    V, D = table.shape
    return _make_tc(B, V, D)(t, table)
